# baseline (speedup 1.0000x reference)
import jax
import jax.numpy as jnp
from jax.experimental import pallas as pl

_LB = 2
_BE = 16


def _argmax_body(x_ref, o_ref):
    lb, c, n = x_ref.shape
    row_iota = jax.lax.broadcasted_iota(jnp.int32, (c, n), 0)
    big = jnp.int32(c)
    for j in range(lb):
        x2 = x_ref[j]
        m = jnp.max(x2, axis=0, keepdims=True)
        idx = jnp.min(jnp.where(x2 == m, row_iota, big), axis=0,
                      keepdims=True)
        o_ref[j] = idx


def _vote_body(i_ref, o_ref):
    r, one = i_ref.shape
    be, c = o_ref.shape
    l = r // be
    idx = i_ref[...]
    iota = jax.lax.broadcasted_iota(jnp.int32, (r, c), 1)
    votes = (iota == idx).astype(jnp.bfloat16)
    seg = (jax.lax.broadcasted_iota(jnp.int32, (be, r), 1) // l
           == jax.lax.broadcasted_iota(jnp.int32, (be, r), 0)
           ).astype(jnp.bfloat16)
    counts = jax.lax.dot_general(
        seg, votes, (((1,), (0,)), ((), ())),
        preferred_element_type=jnp.float32)
    iota_e = jax.lax.broadcasted_iota(jnp.int32, (be, c), 1)
    big = jnp.int32(c)
    m2 = jnp.max(counts, axis=1, keepdims=True)
    win = jnp.min(jnp.where(counts == m2, iota_e, big), axis=1,
                  keepdims=True)
    o_ref[...] = (iota_e == win).astype(jnp.float32)


def kernel(x):
    n, l, c = x.shape
    xt = jnp.transpose(x, (1, 2, 0))
    idx = pl.pallas_call(
        _argmax_body,
        grid=(l // _LB,),
        in_specs=[pl.BlockSpec((_LB, c, n), lambda i: (i, 0, 0))],
        out_specs=pl.BlockSpec((_LB, 1, n), lambda i: (i, 0, 0)),
        out_shape=jax.ShapeDtypeStruct((l, 1, n), jnp.int32),
    )(xt)
    idx_t = jnp.transpose(idx.reshape(l, n), (1, 0)).reshape(n * l, 1)
    return pl.pallas_call(
        _vote_body,
        grid=(n // _BE,),
        in_specs=[pl.BlockSpec((_BE * l, 1), lambda i: (i, 0))],
        out_specs=pl.BlockSpec((_BE, c), lambda i: (i, 0)),
        out_shape=jax.ShapeDtypeStruct((n, c), jnp.float32),
    )(idx_t)


# speedup vs baseline: 1.1938x; 1.1938x over previous
import functools

import jax
import jax.numpy as jnp
from jax import lax
from jax.experimental import pallas as pl
from jax.experimental.pallas import tpu as pltpu
from jax.experimental.pallas import tpu_sc as plsc

_LB = 2
_NW = 32
_CPAD = 1024


def _argmax_body(x_ref, o_ref):
    lb, c, n = x_ref.shape
    row_iota = jax.lax.broadcasted_iota(jnp.int32, (c, n), 0)
    big = jnp.int32(c)
    for j in range(lb):
        x2 = x_ref[j]
        m = jnp.max(x2, axis=0, keepdims=True)
        idx = jnp.min(jnp.where(x2 == m, row_iota, big), axis=0,
                      keepdims=True)
        o_ref[j] = idx


def _sc_vote_body(l, c, ne, idx_hbm, out_hbm, idx_v, counts_v, rows_v):
    wid = lax.axis_index("s") * 2 + lax.axis_index("c")
    iota16 = lax.iota(jnp.int32, 16)
    zeros = jnp.zeros((16,), jnp.float32)
    ones = jnp.ones((16,), jnp.float32)
    pltpu.sync_copy(idx_hbm.at[pl.ds(wid * ne * l, ne * l)], idx_v)

    def zc_body(i, _):
        counts_v[pl.ds(i * 16, 16)] = zeros
        return 0
    lax.fori_loop(0, ne * _CPAD // 16, zc_body, 0)

    def zr_body(i, _):
        rows_v[pl.ds(i * 16, 16)] = zeros
        return 0
    lax.fori_loop(0, ne * c // 16, zr_body, 0)

    def vote_body(t, _):
        for g in range(ne // 16):
            v = idx_v[pl.ds((g * l + t) * 16, 16)]
            tgt = (iota16 + g * 16) * _CPAD + v
            plsc.addupdate_scatter(counts_v, [tgt], ones)
        return 0
    lax.fori_loop(0, l, vote_body, 0)

    def win_body(e, _):
        def m_body(k, m):
            cnt = counts_v[pl.ds(e * _CPAD + k * 16, 16)]
            score = (cnt.astype(jnp.int32) << 16) + (65535 - (k * 16 + iota16))
            return jnp.maximum(m, score)
        m = lax.fori_loop(0, _CPAD // 16, m_body,
                          jnp.full((16,), jnp.int32(-1)))
        best = jnp.max(m, axis=0)
        win = 65535 - (best & 65535)
        mask = iota16 == 0
        plsc.store_scatter(rows_v, [jnp.full((16,), e * c, jnp.int32) + win],
                           ones, mask=mask)
        return 0
    lax.fori_loop(0, ne, win_body, 0)

    pltpu.sync_copy(rows_v, out_hbm.at[pl.ds(wid * ne * c, ne * c)])


def kernel(x):
    n, l, c = x.shape
    xt = jnp.transpose(x, (1, 2, 0))
    idx = pl.pallas_call(
        _argmax_body,
        grid=(l // _LB,),
        in_specs=[pl.BlockSpec((_LB, c, n), lambda i: (i, 0, 0))],
        out_specs=pl.BlockSpec((_LB, 1, n), lambda i: (i, 0, 0)),
        out_shape=jax.ShapeDtypeStruct((l, 1, n), jnp.int32),
    )(xt)
    idx_t = jnp.transpose(idx.reshape(l, n), (1, 0))
    idx_flat = jnp.transpose(idx_t.reshape(n // 16, 16, l),
                             (0, 2, 1)).reshape(n * l)
    ne = n // _NW
    sc_kernel = functools.partial(
        pl.kernel,
        out_type=jax.ShapeDtypeStruct((n * c,), jnp.float32),
        mesh=plsc.VectorSubcoreMesh(core_axis_name="c", subcore_axis_name="s"),
        scratch_types=[
            pltpu.VMEM((ne * l,), jnp.int32),
            pltpu.VMEM((ne * _CPAD,), jnp.float32),
            pltpu.VMEM((ne * c,), jnp.float32),
        ],
        compiler_params=pltpu.CompilerParams(needs_layout_passes=False),
    )(functools.partial(_sc_vote_body, l, c, ne))
    out_flat = sc_kernel(idx_flat)
    return out_flat.reshape(n, c)


# speedup vs baseline: 1.3420x; 1.1242x over previous
import functools

import jax
import jax.numpy as jnp
from jax import lax
from jax.experimental import pallas as pl
from jax.experimental.pallas import tpu as pltpu
from jax.experimental.pallas import tpu_sc as plsc

_LB = 2
_NW = 32
_CPAD = 1024


def _argmax_body(x_ref, o_ref):
    lb, c, n = x_ref.shape
    row_iota = jax.lax.broadcasted_iota(jnp.int32, (c, n), 0)
    big = jnp.int32(c)
    for j in range(lb):
        x2 = x_ref[j]
        m = jnp.max(x2, axis=0, keepdims=True)
        idx = jnp.min(jnp.where(x2 == m, row_iota, big), axis=0,
                      keepdims=True)
        o_ref[j] = idx


def _sc_vote_body(l, c, ne, idx_hbm, out_hbm, idx_v, counts_v, rows_v):
    wid = lax.axis_index("s") * 2 + lax.axis_index("c")
    iota16 = lax.iota(jnp.int32, 16)
    zeros = jnp.zeros((16,), jnp.float32)
    ones = jnp.ones((16,), jnp.float32)
    pltpu.sync_copy(idx_hbm.at[pl.ds(wid * ne * l, ne * l)], idx_v)

    def zc_body(i, _):
        for u in range(8):
            counts_v[pl.ds((i * 8 + u) * 16, 16)] = zeros
        return 0
    lax.fori_loop(0, ne * _CPAD // 128, zc_body, 0)

    def zr_body(i, _):
        for u in range(8):
            rows_v[pl.ds((i * 8 + u) * 16, 16)] = zeros
        return 0
    lax.fori_loop(0, ne * c // 128, zr_body, 0)

    def vote_body(t, _):
        for g in range(ne // 16):
            v = idx_v[pl.ds((g * l + t) * 16, 16)]
            tgt = (iota16 + g * 16) * _CPAD + v
            plsc.addupdate_scatter(counts_v, [tgt], ones)
        return 0
    lax.fori_loop(0, l, vote_body, 0)

    def win_body(e, _):
        def m_body(k, m):
            for u in range(8):
                kk = k * 8 + u
                cnt = counts_v[pl.ds(e * _CPAD + kk * 16, 16)]
                score = ((cnt.astype(jnp.int32) << 16)
                         + (65535 - (kk * 16 + iota16)))
                m = jnp.maximum(m, score)
            return m
        m = lax.fori_loop(0, _CPAD // 128, m_body,
                          jnp.full((16,), jnp.int32(-1)))
        best = jnp.max(m, axis=0)
        win = 65535 - (best & 65535)
        mask = iota16 == 0
        plsc.store_scatter(rows_v, [jnp.full((16,), e * c, jnp.int32) + win],
                           ones, mask=mask)
        return 0
    lax.fori_loop(0, ne, win_body, 0)

    pltpu.sync_copy(rows_v, out_hbm.at[pl.ds(wid * ne * c, ne * c)])


def kernel(x):
    n, l, c = x.shape
    xt = jnp.transpose(x, (1, 2, 0))
    idx = pl.pallas_call(
        _argmax_body,
        grid=(l // _LB,),
        in_specs=[pl.BlockSpec((_LB, c, n), lambda i: (i, 0, 0))],
        out_specs=pl.BlockSpec((_LB, 1, n), lambda i: (i, 0, 0)),
        out_shape=jax.ShapeDtypeStruct((l, 1, n), jnp.int32),
    )(xt)
    idx_t = jnp.transpose(idx.reshape(l, n), (1, 0))
    idx_flat = jnp.transpose(idx_t.reshape(n // 16, 16, l),
                             (0, 2, 1)).reshape(n * l)
    ne = n // _NW
    sc_kernel = functools.partial(
        pl.kernel,
        out_type=jax.ShapeDtypeStruct((n * c,), jnp.float32),
        mesh=plsc.VectorSubcoreMesh(core_axis_name="c", subcore_axis_name="s"),
        scratch_types=[
            pltpu.VMEM((ne * l,), jnp.int32),
            pltpu.VMEM((ne * _CPAD,), jnp.float32),
            pltpu.VMEM((ne * c,), jnp.float32),
        ],
        compiler_params=pltpu.CompilerParams(needs_layout_passes=False),
    )(functools.partial(_sc_vote_body, l, c, ne))
    out_flat = sc_kernel(idx_flat)
    return out_flat.reshape(n, c)


# speedup vs baseline: 1.3893x; 1.0352x over previous
import functools

import jax
import jax.numpy as jnp
from jax import lax
from jax.experimental import pallas as pl
from jax.experimental.pallas import tpu as pltpu
from jax.experimental.pallas import tpu_sc as plsc

_LB = 4
_NW = 32
_CPAD = 1024


def _argmax_body(x_ref, o_ref):
    lb, c, n = x_ref.shape
    row_iota = jax.lax.broadcasted_iota(jnp.int32, (c, n), 0)
    big = jnp.int32(c)
    for j in range(lb):
        x2 = x_ref[j]
        m = jnp.max(x2, axis=0, keepdims=True)
        idx = jnp.min(jnp.where(x2 == m, row_iota, big), axis=0,
                      keepdims=True)
        o_ref[j] = idx


def _sc_vote_body(l, c, ne, idx_hbm, out_hbm, idx_v, counts_v, rows_v):
    wid = lax.axis_index("s") * 2 + lax.axis_index("c")
    iota16 = lax.iota(jnp.int32, 16)
    zeros = jnp.zeros((16,), jnp.float32)
    ones = jnp.ones((16,), jnp.float32)
    pltpu.sync_copy(idx_hbm.at[pl.ds(wid * ne * l, ne * l)], idx_v)

    def zc_body(i, _):
        for u in range(8):
            counts_v[pl.ds((i * 8 + u) * 16, 16)] = zeros
        return 0
    lax.fori_loop(0, ne * _CPAD // 128, zc_body, 0)

    def zr_body(i, _):
        for u in range(8):
            rows_v[pl.ds((i * 8 + u) * 16, 16)] = zeros
        return 0
    lax.fori_loop(0, ne * c // 128, zr_body, 0)

    def vote_body(t, _):
        for g in range(ne // 16):
            v = idx_v[pl.ds((g * l + t) * 16, 16)]
            tgt = (iota16 + g * 16) * _CPAD + v
            plsc.addupdate_scatter(counts_v, [tgt], ones)
        return 0
    lax.fori_loop(0, l, vote_body, 0)

    def win_body(e, _):
        def m_body(k, m):
            for u in range(8):
                kk = k * 8 + u
                cnt = counts_v[pl.ds(e * _CPAD + kk * 16, 16)]
                score = ((cnt.astype(jnp.int32) << 16)
                         + (65535 - (kk * 16 + iota16)))
                m = jnp.maximum(m, score)
            return m
        m = lax.fori_loop(0, _CPAD // 128, m_body,
                          jnp.full((16,), jnp.int32(-1)))
        best = jnp.max(m, axis=0)
        win = 65535 - (best & 65535)
        mask = iota16 == 0
        plsc.store_scatter(rows_v, [jnp.full((16,), e * c, jnp.int32) + win],
                           ones, mask=mask)
        return 0
    lax.fori_loop(0, ne, win_body, 0)

    pltpu.sync_copy(rows_v, out_hbm.at[pl.ds(wid * ne * c, ne * c)])


def kernel(x):
    n, l, c = x.shape
    xt = jnp.transpose(x, (1, 2, 0))
    idx = pl.pallas_call(
        _argmax_body,
        grid=(l // _LB,),
        in_specs=[pl.BlockSpec((_LB, c, n), lambda i: (i, 0, 0))],
        out_specs=pl.BlockSpec((_LB, 1, n), lambda i: (i, 0, 0)),
        out_shape=jax.ShapeDtypeStruct((l, 1, n), jnp.int32),
    )(xt)
    idx_t = jnp.transpose(idx.reshape(l, n), (1, 0))
    idx_flat = jnp.transpose(idx_t.reshape(n // 16, 16, l),
                             (0, 2, 1)).reshape(n * l)
    ne = n // _NW
    sc_kernel = functools.partial(
        pl.kernel,
        out_type=jax.ShapeDtypeStruct((n * c,), jnp.float32),
        mesh=plsc.VectorSubcoreMesh(core_axis_name="c", subcore_axis_name="s"),
        scratch_types=[
            pltpu.VMEM((ne * l,), jnp.int32),
            pltpu.VMEM((ne * _CPAD,), jnp.float32),
            pltpu.VMEM((ne * c,), jnp.float32),
        ],
        compiler_params=pltpu.CompilerParams(needs_layout_passes=False),
    )(functools.partial(_sc_vote_body, l, c, ne))
    out_flat = sc_kernel(idx_flat)
    return out_flat.reshape(n, c)
